# baseline (speedup 1.0000x reference)
import functools

import jax
import jax.numpy as jnp
from jax import lax
from jax.experimental import pallas as pl
from jax.experimental.pallas import tpu as pltpu
from jax.experimental.pallas import tpu_sc as plsc

_E = 8
_T = 16384
_NW = 32
_TPW = _T // _NW
_LANES = 16


def _route_rows(s_rows):
    f32 = s_rows[0].dtype
    i32 = jnp.int32
    g = [jnp.maximum(s_rows[2 * k], s_rows[2 * k + 1]) for k in range(4)]
    m1 = jnp.maximum(jnp.maximum(g[0], g[1]), jnp.maximum(g[2], g[3]))
    gi1 = jnp.where(
        g[0] == m1, 0,
        jnp.where(g[1] == m1, 1, jnp.where(g[2] == m1, 2, 3))).astype(i32)
    ge = [jnp.where(gi1 == k, jnp.asarray(-1.0, f32), g[k]) for k in range(4)]
    m2 = jnp.maximum(jnp.maximum(ge[0], ge[1]), jnp.maximum(ge[2], ge[3]))
    gi2 = jnp.where(
        ge[0] == m2, 0,
        jnp.where(ge[1] == m2, 1, jnp.where(ge[2] == m2, 2, 3))).astype(i32)
    ms = [
        jnp.where((gi1 == (e // 2)) | (gi2 == (e // 2)), s_rows[e],
                  jnp.asarray(0.0, f32)) for e in range(8)
    ]
    M1 = ms[0]
    for e in range(1, 8):
        M1 = jnp.maximum(M1, ms[e])
    e1 = jnp.full_like(gi1, 7)
    for e in range(6, -1, -1):
        e1 = jnp.where(ms[e] == M1, e, e1).astype(i32)
    mse = [jnp.where(e1 == e, jnp.asarray(-1.0, f32), ms[e]) for e in range(8)]
    M2 = mse[0]
    for e in range(1, 8):
        M2 = jnp.maximum(M2, mse[e])
    e2 = jnp.full_like(gi1, 7)
    for e in range(6, -1, -1):
        e2 = jnp.where(mse[e] == M2, e, e2).astype(i32)
    denom = M1 + M2 + jnp.asarray(1e-20, f32)
    return e1, e2, M1 / denom, M2 / denom


def _logits_block(x_ref, w_ref, lt_ref):
    lt_ref[...] = jax.lax.dot_general(
        w_ref[...], x_ref[...], (((1,), (1,)), ((), ())),
        preferred_element_type=jnp.float32)


@functools.partial(jax.jit, static_argnames=("block_t",))
def _logits_tc(x, weight, block_t=1024):
    t, h = x.shape
    return pl.pallas_call(
        _logits_block,
        grid=(t // block_t,),
        in_specs=[
            pl.BlockSpec((block_t, h), lambda i: (i, 0)),
            pl.BlockSpec((weight.shape[0], h), lambda i: (0, 0)),
        ],
        out_specs=pl.BlockSpec((weight.shape[0], block_t), lambda i: (0, i)),
        out_shape=jax.ShapeDtypeStruct((weight.shape[0], t), jnp.float32),
    )(x, weight)


def _route_body(lt_hbm, idx_hbm, wgt_hbm, lt_v, idx_v, wgt_v):
    return
    wid = lax.axis_index("s") * 2 + lax.axis_index("c")
    base = wid * _TPW
    for e in range(_E):
        pltpu.sync_copy(lt_hbm.at[e, pl.ds(base, _TPW)], lt_v.at[e])

    def step(j, carry):
        o = j * _LANES
        l_ = [lt_v[e, pl.ds(o, _LANES)] for e in range(_E)]
        m = l_[0]
        for e in range(1, _E):
            m = jnp.maximum(m, l_[e])
        ex = [jnp.exp(v - m) for v in l_]
        z = ex[0]
        for e in range(1, _E):
            z = z + ex[e]
        rinv = jnp.asarray(1.0, jnp.float32) / z
        s = [v * rinv for v in ex]
        e1, e2, w1, w2 = _route_rows(s)
        idx_v[0, pl.ds(o, _LANES)] = e1
        idx_v[1, pl.ds(o, _LANES)] = e2
        wgt_v[0, pl.ds(o, _LANES)] = w1
        wgt_v[1, pl.ds(o, _LANES)] = w2
        return carry

    lax.fori_loop(0, _TPW // _LANES, step, 0)
    for r in range(2):
        pltpu.sync_copy(idx_v.at[r], idx_hbm.at[r, pl.ds(base, _TPW)])
        pltpu.sync_copy(wgt_v.at[r], wgt_hbm.at[r, pl.ds(base, _TPW)])


_route_sc = functools.partial(
    pl.kernel,
    mesh=plsc.VectorSubcoreMesh(core_axis_name="c", subcore_axis_name="s"),
    out_type=[
        jax.ShapeDtypeStruct((2, _T), jnp.int32),
        jax.ShapeDtypeStruct((2, _T), jnp.float32),
    ],
    scratch_types=[
        pltpu.VMEM((_E, _TPW), jnp.float32),
        pltpu.VMEM((2, _TPW), jnp.int32),
        pltpu.VMEM((2, _TPW), jnp.float32),
    ],
)(_route_body)


def kernel(hidden_states, weight):
    bsz, seq_len, h = hidden_states.shape
    x = hidden_states.reshape(-1, h)
    lt = _logits_tc(x, weight)
    idx_t, wgt_t = _route_sc(lt)
    return idx_t.T, wgt_t.T


# speedup vs baseline: 1.3679x; 1.3679x over previous
import functools

import jax
import jax.numpy as jnp
from jax.experimental import pallas as pl

_E = 8


def _select_top2(l_rows):
    f32 = l_rows[0].dtype
    i32 = jnp.int32
    ninf = jnp.asarray(-jnp.inf, f32)
    g = [jnp.maximum(l_rows[2 * k], l_rows[2 * k + 1]) for k in range(4)]
    m1 = jnp.maximum(jnp.maximum(g[0], g[1]), jnp.maximum(g[2], g[3]))
    gi1 = jnp.where(
        g[0] == m1, 0,
        jnp.where(g[1] == m1, 1, jnp.where(g[2] == m1, 2, 3))).astype(i32)
    ge = [jnp.where(gi1 == k, ninf, g[k]) for k in range(4)]
    m2 = jnp.maximum(jnp.maximum(ge[0], ge[1]), jnp.maximum(ge[2], ge[3]))
    gi2 = jnp.where(
        ge[0] == m2, 0,
        jnp.where(ge[1] == m2, 1, jnp.where(ge[2] == m2, 2, 3))).astype(i32)
    keep = [(gi1 == k) | (gi2 == k) for k in range(4)]
    ms = [jnp.where(keep[e // 2], l_rows[e], ninf) for e in range(8)]
    M1 = ms[0]
    for e in range(1, 8):
        M1 = jnp.maximum(M1, ms[e])
    e1 = jnp.full_like(gi1, 7)
    for e in range(6, -1, -1):
        e1 = jnp.where(ms[e] == M1, e, e1).astype(i32)
    mse = [jnp.where(e1 == e, ninf, ms[e]) for e in range(8)]
    M2 = mse[0]
    for e in range(1, 8):
        M2 = jnp.maximum(M2, mse[e])
    e2 = jnp.full_like(gi1, 7)
    for e in range(6, -1, -1):
        e2 = jnp.where(mse[e] == M2, e, e2).astype(i32)
    return e1, e2, M1, M2


def _gate_block(x_ref, w_ref, idx_ref, wgt_ref):
    lt = jax.lax.dot_general(w_ref[...], x_ref[...], (((1,), (1,)), ((), ())),
                             preferred_element_type=jnp.float32)
    rows = [lt[e:e + 1, :] for e in range(_E)]
    e1, e2, l1, l2 = _select_top2(rows)
    e21 = jnp.exp(l2 - l1)
    q = jnp.asarray(1.0, jnp.float32) / (jnp.asarray(1.0, jnp.float32) + e21)
    idx_ref[...] = jnp.concatenate([e1, e2], axis=0)
    wgt_ref[...] = jnp.concatenate([q, e21 * q], axis=0)


@functools.partial(jax.jit, static_argnames=("block_t",))
def _moe_gate_tc(x, weight, block_t=1024):
    t, h = x.shape
    idx_t, wgt_t = pl.pallas_call(
        _gate_block,
        grid=(t // block_t,),
        in_specs=[
            pl.BlockSpec((block_t, h), lambda i: (i, 0)),
            pl.BlockSpec((weight.shape[0], h), lambda i: (0, 0)),
        ],
        out_specs=[
            pl.BlockSpec((2, block_t), lambda i: (0, i)),
            pl.BlockSpec((2, block_t), lambda i: (0, i)),
        ],
        out_shape=[
            jax.ShapeDtypeStruct((2, t), jnp.int32),
            jax.ShapeDtypeStruct((2, t), jnp.float32),
        ],
    )(x, weight)
    return idx_t.T, wgt_t.T


def kernel(hidden_states, weight):
    bsz, seq_len, h = hidden_states.shape
    x = hidden_states.reshape(-1, h)
    return _moe_gate_tc(x, weight)
